# baseline (speedup 1.0000x reference)
import functools

import jax
import jax.numpy as jnp
from jax import lax
from jax.experimental import pallas as pl
from jax.experimental.pallas import tpu as pltpu

_FM_H = 19
_FM_W = 19
_HW = _FM_H * _FM_W
_SCORE_FILTER_SIZE = (3, 2)
_SCORE_FILTER_NUM = (3, 3)
_PATCH_SIZE = (224, 112)
_NEG = float("-inf")
_NBUF = 4


def _shift(x, o, fill):
    if o == 0:
        return x
    n = x.shape[1]
    f = jnp.full((x.shape[0], abs(o)), fill, x.dtype)
    if o > 0:
        return jnp.concatenate([x[:, o:], f], axis=1)
    return jnp.concatenate([f, x[:, : n + o]], axis=1)


def _batched_nms(s_all, scale_ref, loc_ref, val_ref):
    bsz = s_all.shape[0]
    p_i = lax.broadcasted_iota(jnp.int32, (bsz, _HW), 1)
    row = p_i // _FM_W
    col = p_i % _FM_W

    sh = scale_ref[:, 0:1]
    sw = scale_ref[:, 1:2]
    smin = jnp.minimum(sh, sw)
    sb_hf = ((sh - smin) // 2).astype(jnp.float32)
    sb_wf = ((sw - smin) // 2).astype(jnp.float32)
    smin_f = smin.astype(jnp.float32)

    loc_cols = []
    val_cols = []
    for t in range(2):
        k = _SCORE_FILTER_SIZE[t]
        ps = _PATCH_SIZE[t]
        s = _FM_H - k + 1
        pooled = jnp.zeros((bsz, _HW), jnp.float32)
        for di in range(k):
            for dj in range(k):
                pooled = pooled + _shift(s_all, di * _FM_W + dj, 0.0)
        pooled = pooled / float(k * k)
        valid = (row < s) & (col < s)
        score = jnp.where(valid, pooled, _NEG)
        for _f in range(_SCORE_FILTER_NUM[t]):
            vmax = jnp.max(score, axis=1, keepdims=True)
            psel = jnp.min(jnp.where(score == vmax, p_i, _HW),
                           axis=1, keepdims=True)
            tp = jnp.full((bsz, _HW), _NEG, jnp.float32)
            for di in (-1, 0, 1):
                for dj in (-1, 0, 1):
                    nb_ = _shift(score, di * _FM_W + dj, _NEG)
                    okc = (col + dj >= 0) & (col + dj < _FM_W)
                    tp = jnp.maximum(tp, jnp.where(okc, nb_, _NEG))
            score = jnp.where((tp == vmax) & valid, 0.0, score)

            i = psel // _FM_W
            j = psel % _FM_W
            rate_h = (2.0 * i.astype(jnp.float32) + float(_FM_H - s + 1)) / (2.0 * _FM_H)
            rate_w = (2.0 * j.astype(jnp.float32) + float(_FM_W - s + 1)) / (2.0 * _FM_W)
            c_h = (sb_hf + smin_f * rate_h).astype(jnp.int32)
            c_w = (sb_wf + smin_f * rate_w).astype(jnp.int32)
            top = c_h - ps // 2
            bot = c_h + ps // 2 + ps % 2
            lef = c_w - ps // 2
            rig = c_w + ps // 2 + ps % 2
            below_h = jnp.minimum(top, 0)
            top = top - below_h
            bot = bot - below_h
            below_w = jnp.minimum(lef, 0)
            lef = lef - below_w
            rig = rig - below_w
            over_h = jnp.maximum(bot - sh, 0)
            top = jnp.maximum(top - over_h, 0)
            bot = bot - over_h
            over_w = jnp.maximum(rig - sw, 0)
            lef = jnp.maximum(lef - over_w, 0)
            rig = rig - over_w
            loc_cols.append(jnp.concatenate([top, lef, bot, rig], axis=1))
            val_cols.append(vmax)

    loc_ref[...] = jnp.concatenate(loc_cols, axis=1)
    val_ref[...] = jnp.concatenate(val_cols, axis=1)


def _body(fm_hbm, scale_ref, loc_ref, val_ref, bufs, sems, s_scratch,
          *, r_chunk):
    bsz = fm_hbm.shape[0]
    n_rows, n_lanes = fm_hbm.shape[1], fm_hbm.shape[2]
    n_fold = n_lanes // _HW
    n_chunks = n_rows // r_chunk

    def dma(i, slot):
        return pltpu.make_async_copy(fm_hbm.at[i], bufs.at[slot],
                                     sems.at[slot])

    for slot in range(_NBUF):
        dma(slot, slot).start()

    def group(g, _):
        for slot in range(_NBUF):
            i = g * _NBUF + slot
            dma(i, slot).wait()

            def red(cc, acc):
                return acc + bufs[slot, pl.ds(cc * r_chunk, r_chunk), :]

            acc = lax.fori_loop(0, n_chunks, red,
                                jnp.zeros((r_chunk, n_lanes), jnp.float32))
            acc1 = jnp.sum(acc, axis=0, keepdims=True)
            s361 = acc1[:, 0:_HW]
            for fj in range(1, n_fold):
                s361 = s361 + acc1[:, fj * _HW:(fj + 1) * _HW]
            s_scratch[pl.ds(i, 1), :] = s361

            nxt = i + _NBUF

            @pl.when(nxt < bsz)
            def _():
                dma(nxt, slot).start()
        return 0

    lax.fori_loop(0, bsz // _NBUF, group, 0)
    _batched_nms(s_scratch[...], scale_ref, loc_ref, val_ref)


@jax.jit
def kernel(fm, scale):
    b, c, h, w = fm.shape
    fm2 = fm.reshape(b, c, h * w)

    loc, vals = pl.pallas_call(
        functools.partial(_body, r_chunk=8),
        in_specs=[
            pl.BlockSpec(memory_space=pl.ANY),
            pl.BlockSpec(memory_space=pltpu.MemorySpace.VMEM),
        ],
        out_specs=[
            pl.BlockSpec(memory_space=pltpu.MemorySpace.VMEM),
            pl.BlockSpec(memory_space=pltpu.MemorySpace.VMEM),
        ],
        out_shape=[
            jax.ShapeDtypeStruct((b, 24), jnp.int32),
            jax.ShapeDtypeStruct((b, 6), jnp.float32),
        ],
        scratch_shapes=[
            pltpu.VMEM((_NBUF, c, h * w), jnp.float32),
            pltpu.SemaphoreType.DMA((_NBUF,)),
            pltpu.VMEM((b, _HW), jnp.float32),
        ],
    )(fm2, scale)
    return loc.reshape(b, 6, 4), vals.reshape(b, 6)


# speedup vs baseline: 1.1525x; 1.1525x over previous
import functools

import jax
import jax.numpy as jnp
from jax import lax
from jax.experimental import pallas as pl
from jax.experimental.pallas import tpu as pltpu

_FM_H = 19
_FM_W = 19
_HW = _FM_H * _FM_W
_SCORE_FILTER_SIZE = (3, 2)
_SCORE_FILTER_NUM = (3, 3)
_PATCH_SIZE = (224, 112)
_NEG = float("-inf")
_NBUF = 4


def _shift(x, o, fill):
    if o == 0:
        return x
    n = x.shape[1]
    f = jnp.full((x.shape[0], abs(o)), fill, x.dtype)
    if o > 0:
        return jnp.concatenate([x[:, o:], f], axis=1)
    return jnp.concatenate([f, x[:, : n + o]], axis=1)


def _batched_nms(s_all, scale_ref, loc_ref, val_ref):
    bsz = s_all.shape[0]
    p_i = lax.broadcasted_iota(jnp.int32, (bsz, _HW), 1)
    row = p_i // _FM_W
    col = p_i % _FM_W

    sh = scale_ref[:, 0:1]
    sw = scale_ref[:, 1:2]
    smin = jnp.minimum(sh, sw)
    sb_hf = ((sh - smin) // 2).astype(jnp.float32)
    sb_wf = ((sw - smin) // 2).astype(jnp.float32)
    smin_f = smin.astype(jnp.float32)

    loc_cols = []
    val_cols = []
    for t in range(2):
        k = _SCORE_FILTER_SIZE[t]
        ps = _PATCH_SIZE[t]
        s = _FM_H - k + 1
        pooled = jnp.zeros((bsz, _HW), jnp.float32)
        for di in range(k):
            for dj in range(k):
                pooled = pooled + _shift(s_all, di * _FM_W + dj, 0.0)
        pooled = pooled / float(k * k)
        valid = (row < s) & (col < s)
        score = jnp.where(valid, pooled, _NEG)
        for _f in range(_SCORE_FILTER_NUM[t]):
            vmax = jnp.max(score, axis=1, keepdims=True)
            psel = jnp.min(jnp.where(score == vmax, p_i, _HW),
                           axis=1, keepdims=True)
            tp = jnp.full((bsz, _HW), _NEG, jnp.float32)
            for di in (-1, 0, 1):
                for dj in (-1, 0, 1):
                    nb_ = _shift(score, di * _FM_W + dj, _NEG)
                    okc = (col + dj >= 0) & (col + dj < _FM_W)
                    tp = jnp.maximum(tp, jnp.where(okc, nb_, _NEG))
            score = jnp.where((tp == vmax) & valid, 0.0, score)

            i = psel // _FM_W
            j = psel % _FM_W
            rate_h = (2.0 * i.astype(jnp.float32) + float(_FM_H - s + 1)) / (2.0 * _FM_H)
            rate_w = (2.0 * j.astype(jnp.float32) + float(_FM_W - s + 1)) / (2.0 * _FM_W)
            c_h = (sb_hf + smin_f * rate_h).astype(jnp.int32)
            c_w = (sb_wf + smin_f * rate_w).astype(jnp.int32)
            top = c_h - ps // 2
            bot = c_h + ps // 2 + ps % 2
            lef = c_w - ps // 2
            rig = c_w + ps // 2 + ps % 2
            below_h = jnp.minimum(top, 0)
            top = top - below_h
            bot = bot - below_h
            below_w = jnp.minimum(lef, 0)
            lef = lef - below_w
            rig = rig - below_w
            over_h = jnp.maximum(bot - sh, 0)
            top = jnp.maximum(top - over_h, 0)
            bot = bot - over_h
            over_w = jnp.maximum(rig - sw, 0)
            lef = jnp.maximum(lef - over_w, 0)
            rig = rig - over_w
            loc_cols.append(jnp.concatenate([top, lef, bot, rig], axis=1))
            val_cols.append(vmax)

    loc_ref[...] = jnp.concatenate(loc_cols, axis=1)
    val_ref[...] = jnp.concatenate(val_cols, axis=1)


def _body(fm_hbm, scale_ref, loc_ref, val_ref, bufs, sems, s_scratch,
          *, r_chunk):
    bsz = fm_hbm.shape[0]
    n_rows, n_lanes = fm_hbm.shape[1], fm_hbm.shape[2]
    n_fold = n_lanes // _HW
    n_chunks = n_rows // r_chunk

    def dma(i, slot):
        return pltpu.make_async_copy(fm_hbm.at[i], bufs.at[slot],
                                     sems.at[slot])

    for slot in range(_NBUF):
        dma(slot, slot).start()

    def group(g, _):
        for slot in range(_NBUF):
            i = g * _NBUF + slot
            dma(i, slot).wait()

            def red(cc, acc):
                return acc + bufs[slot, pl.ds(cc * r_chunk, r_chunk), :]

            acc = lax.fori_loop(0, n_chunks, red,
                                jnp.zeros((r_chunk, n_lanes), jnp.float32))
            acc1 = jnp.sum(acc, axis=0, keepdims=True)
            s361 = acc1[:, 0:_HW]
            for fj in range(1, n_fold):
                s361 = s361 + acc1[:, fj * _HW:(fj + 1) * _HW]
            s_scratch[pl.ds(i, 1), :] = s361

            nxt = i + _NBUF

            @pl.when(nxt < bsz)
            def _():
                dma(nxt, slot).start()
        return 0

    lax.fori_loop(0, bsz // _NBUF, group, 0)
    _batched_nms(s_scratch[...], scale_ref, loc_ref, val_ref)


@jax.jit
def kernel(fm, scale):
    b, c, h, w = fm.shape
    fm2 = fm.reshape(b, c, h * w)

    loc, vals = pl.pallas_call(
        functools.partial(_body, r_chunk=64),
        in_specs=[
            pl.BlockSpec(memory_space=pl.ANY),
            pl.BlockSpec(memory_space=pltpu.MemorySpace.VMEM),
        ],
        out_specs=[
            pl.BlockSpec(memory_space=pltpu.MemorySpace.VMEM),
            pl.BlockSpec(memory_space=pltpu.MemorySpace.VMEM),
        ],
        out_shape=[
            jax.ShapeDtypeStruct((b, 24), jnp.int32),
            jax.ShapeDtypeStruct((b, 6), jnp.float32),
        ],
        scratch_shapes=[
            pltpu.VMEM((_NBUF, c, h * w), jnp.float32),
            pltpu.SemaphoreType.DMA((_NBUF,)),
            pltpu.VMEM((b, _HW), jnp.float32),
        ],
    )(fm2, scale)
    return loc.reshape(b, 6, 4), vals.reshape(b, 6)
